# baseline (speedup 1.0000x reference)
import functools

import jax
import jax.numpy as jnp
from jax import lax
from jax.experimental import pallas as pl
from jax.experimental.pallas import tpu as pltpu
from jax.experimental.pallas import tpu_sc as plsc

B = 16384
D = 64
NE = 1000000
NW = 32
BW = B // NW
NCHUNK = 4
CW = BW // NCHUNK
L = 16
W = 32768
E_SC = 4 * W
CPW = E_SC // NW
CH = 512
NCH = CPW // CH
TC_GRID = -(-(NE - E_SC) // W)


def _norms_body(x_ref, o_ref):
    x = x_ref[...]
    for w in range(o_ref.shape[0]):
        sq = x[:, w * 128:(w + 1) * 128]
        o_ref[pl.ds(w, 1), :] = jnp.sum(sq * sq, axis=0, keepdims=True)


def _tc_entity_norms(tt):
    return pl.pallas_call(
        _norms_body,
        grid=(TC_GRID,),
        in_specs=[pl.BlockSpec((D, W), lambda c: (0, c + E_SC // W))],
        out_specs=pl.BlockSpec((W // 128, 128), lambda c: (c, 0)),
        out_shape=jax.ShapeDtypeStruct((TC_GRID * W // 128, 128),
                                       jnp.float32),
    )(tt)


def _tc_relation_norms(rt_pad):
    return pl.pallas_call(
        _norms_body,
        out_shape=jax.ShapeDtypeStruct((8, 128), jnp.float32),
    )(rt_pad)


def _sc_dense_norms(tt):
    mesh = plsc.VectorSubcoreMesh(core_axis_name="c", subcore_axis_name="s")

    @functools.partial(
        pl.kernel,
        mesh=mesh,
        out_type=jax.ShapeDtypeStruct((E_SC,), jnp.float32),
        compiler_params=pltpu.CompilerParams(needs_layout_passes=False),
        scratch_types=[
            [pltpu.VMEM((D, CH), jnp.float32) for _ in range(2)],
            pltpu.VMEM((CPW,), jnp.float32),
            [pltpu.SemaphoreType.DMA for _ in range(2)],
        ],
    )
    def sc_kernel(tt_hbm, out_hbm, bufs, sbuf, sems):
        wid = lax.axis_index("s") * 2 + lax.axis_index("c")
        base = wid * CPW
        handles = [None] * NCH

        def fire(c):
            b = c % 2
            handles[c] = pltpu.async_copy(
                tt_hbm.at[pl.ds(0, D), pl.ds(base + c * CH, CH)],
                bufs[b], sems[b])

        fire(0)
        for c in range(NCH):
            handles[c].wait()
            if c + 1 < NCH:
                fire(c + 1)
            buf = bufs[c % 2]

            def body(g, carry, _buf=buf, _c=c):
                sl = pl.ds(g * L, L)

                def jbody(j, accs):
                    a0, a1, a2, a3 = accs
                    v0 = _buf[4 * j, sl]
                    v1 = _buf[4 * j + 1, sl]
                    v2 = _buf[4 * j + 2, sl]
                    v3 = _buf[4 * j + 3, sl]
                    return (a0 + v0 * v0, a1 + v1 * v1,
                            a2 + v2 * v2, a3 + v3 * v3)

                z = jnp.zeros((L,), jnp.float32)
                a0, a1, a2, a3 = lax.fori_loop(0, D // 4, jbody,
                                               (z, z, z, z), unroll=8)
                sbuf[pl.ds(_c * CH + g * L, L)] = (a0 + a1) + (a2 + a3)
                return carry

            lax.fori_loop(0, CH // L, body, 0)

        pltpu.sync_copy(sbuf, out_hbm.at[pl.ds(base, CPW)])

    return sc_kernel(tt)


def _sc_gather(h, t, r, s_lo, s_hi, sr1d):
    mesh = plsc.VectorSubcoreMesh(core_axis_name="c", subcore_axis_name="s")

    @functools.partial(
        pl.kernel,
        mesh=mesh,
        out_type=jax.ShapeDtypeStruct((3 * B,), jnp.float32),
        compiler_params=pltpu.CompilerParams(needs_layout_passes=False),
        scratch_types=[
            [pltpu.VMEM((BW,), jnp.int32) for _ in range(3)],
            [pltpu.VMEM((BW,), jnp.int32) for _ in range(2)],
            [pltpu.VMEM((BW,), jnp.int32) for _ in range(2)],
            [pltpu.VMEM((BW,), jnp.float32) for _ in range(3)],
            [pltpu.VMEM((BW,), jnp.float32) for _ in range(2)],
            pltpu.VMEM((1024,), jnp.float32),
            pltpu.SemaphoreType.DMA,
        ],
    )
    def sc_kernel(h_hbm, t_hbm, r_hbm, lo_hbm, hi_hbm, sr_hbm, out_hbm,
                  raws, los, his, vals, valhis, srv, sem):
        wid = lax.axis_index("s") * 2 + lax.axis_index("c")
        base = wid * BW

        for k, src in enumerate((h_hbm, t_hbm, r_hbm)):
            pltpu.sync_copy(src.at[pl.ds(base, BW)], raws[k])
        pltpu.sync_copy(sr_hbm, srv)

        for k in range(2):
            for g in range(BW // L):
                sl = pl.ds(g * L, L)
                v = raws[k][sl]
                los[k][sl] = jnp.minimum(v, E_SC - 1)
                his[k][sl] = jnp.maximum(v - E_SC, 0)

        copies = []
        for k in range(2):
            for c in range(NCHUNK):
                sl = pl.ds(c * CW, CW)
                copies.append(pltpu.async_copy(
                    lo_hbm.at[los[k].at[sl]], vals[k].at[sl], sem))
                copies.append(pltpu.async_copy(
                    hi_hbm.at[his[k].at[sl]], valhis[k].at[sl], sem))

        for g in range(BW // L):
            sl = pl.ds(g * L, L)
            vals[2][sl] = plsc.load_gather(srv, [raws[2][sl]])

        for cp in copies:
            cp.wait()

        for k in range(2):
            for g in range(BW // L):
                sl = pl.ds(g * L, L)
                m = raws[k][sl] < E_SC
                vals[k][sl] = jnp.where(m, vals[k][sl], valhis[k][sl])

        for k in range(3):
            pltpu.sync_copy(vals[k], out_hbm.at[pl.ds(k * B + base, BW)])

    return sc_kernel(h, t, r, s_lo, s_hi, sr1d)


def _tc_combine(p):
    def body(p_ref, o_ref):
        n = p_ref.shape[0] // 3
        d = (jnp.sqrt(p_ref[0:n, :]) - jnp.sqrt(p_ref[n:2 * n, :])
             + jnp.sqrt(p_ref[2 * n:3 * n, :]))
        o_ref[...] = jnp.sqrt(jnp.sum(d * d)).reshape(1, 1)

    return pl.pallas_call(
        body,
        out_shape=jax.ShapeDtypeStruct((1, 1), jnp.float32),
    )(p)


def kernel(h, r, t, emb_entity, emb_relation, norm_p):
    tt = emb_entity.T
    s_lo = _sc_dense_norms(tt)
    s_hi = _tc_entity_norms(tt).reshape(-1)
    sr = _tc_relation_norms(jnp.pad(emb_relation.T, ((0, 0), (0, 24))))
    p = _sc_gather(h, t, r, s_lo, s_hi, sr.reshape(-1))
    out = _tc_combine(p.reshape(3 * B // 128, 128))[0, 0]
    pf = jnp.asarray(norm_p, jnp.float32)
    return out * (pf / pf)


# speedup vs baseline: 2.2747x; 2.2747x over previous
import functools

import jax
import jax.numpy as jnp
from jax import lax
from jax.experimental import pallas as pl
from jax.experimental.pallas import tpu as pltpu
from jax.experimental.pallas import tpu_sc as plsc

B = 16384
D = 64
NE = 1000000
NW = 32
BW = B // NW
NCHUNK = 4
CW = BW // NCHUNK
L = 16
W = 32768
GRID = -(-NE // W)


def _norms_body(x_ref, o_ref):
    x = x_ref[...]
    for w in range(o_ref.shape[0]):
        sq = x[:, w * 128:(w + 1) * 128]
        o_ref[pl.ds(w, 1), :] = jnp.sum(sq * sq, axis=0, keepdims=True)


def _tc_entity_norms(tt):
    return pl.pallas_call(
        _norms_body,
        grid=(GRID,),
        in_specs=[pl.BlockSpec((D, W), lambda c: (0, c))],
        out_specs=pl.BlockSpec((W // 128, 128), lambda c: (c, 0)),
        out_shape=jax.ShapeDtypeStruct((GRID * W // 128, 128), jnp.float32),
    )(tt)


def _tc_relation_norms(rt_pad):
    return pl.pallas_call(
        _norms_body,
        out_shape=jax.ShapeDtypeStruct((8, 128), jnp.float32),
    )(rt_pad)


def _sc_gather(h, t, r, s1d, sr1d):
    mesh = plsc.VectorSubcoreMesh(core_axis_name="c", subcore_axis_name="s")

    @functools.partial(
        pl.kernel,
        mesh=mesh,
        out_type=jax.ShapeDtypeStruct((3 * B,), jnp.float32),
        compiler_params=pltpu.CompilerParams(needs_layout_passes=False),
        scratch_types=[
            [pltpu.VMEM((BW,), jnp.int32) for _ in range(3)],
            [pltpu.VMEM((BW,), jnp.float32) for _ in range(3)],
            pltpu.VMEM((1024,), jnp.float32),
            pltpu.SemaphoreType.DMA,
        ],
    )
    def sc_kernel(h_hbm, t_hbm, r_hbm, s_hbm, sr_hbm, out_hbm,
                  raws, vals, srv, sem):
        wid = lax.axis_index("s") * 2 + lax.axis_index("c")
        base = wid * BW

        for k, src in enumerate((h_hbm, t_hbm, r_hbm)):
            pltpu.sync_copy(src.at[pl.ds(base, BW)], raws[k])
        pltpu.sync_copy(sr_hbm, srv)

        copies = []
        for k in range(2):
            for c in range(NCHUNK):
                sl = pl.ds(c * CW, CW)
                copies.append(pltpu.async_copy(
                    s_hbm.at[raws[k].at[sl]], vals[k].at[sl], sem))

        for g in range(BW // L):
            v = raws[2][pl.ds(g * L, L)]
            vals[2][pl.ds(g * L, L)] = plsc.load_gather(srv, [v])

        for cp in copies:
            cp.wait()
        for k in range(3):
            pltpu.sync_copy(vals[k], out_hbm.at[pl.ds(k * B + base, BW)])

    return sc_kernel(h, t, r, s1d, sr1d)


def _tc_combine(p):
    def body(p_ref, o_ref):
        n = p_ref.shape[0] // 3
        d = (jnp.sqrt(p_ref[0:n, :]) - jnp.sqrt(p_ref[n:2 * n, :])
             + jnp.sqrt(p_ref[2 * n:3 * n, :]))
        o_ref[...] = jnp.sqrt(jnp.sum(d * d)).reshape(1, 1)

    return pl.pallas_call(
        body,
        out_shape=jax.ShapeDtypeStruct((1, 1), jnp.float32),
    )(p)


def kernel(h, r, t, emb_entity, emb_relation, norm_p):
    s2 = _tc_entity_norms(emb_entity.T)
    sr = _tc_relation_norms(jnp.pad(emb_relation.T, ((0, 0), (0, 24))))
    p = _sc_gather(h, t, r, s2.reshape(-1), sr.reshape(-1))
    out = _tc_combine(p.reshape(3 * B // 128, 128))[0, 0]
    pf = jnp.asarray(norm_p, jnp.float32)
    return out * (pf / pf)
